# baseline (speedup 1.0000x reference)
import functools

import jax
import jax.numpy as jnp
from jax import lax
from jax.experimental import pallas as pl
from jax.experimental.pallas import tpu as pltpu
from jax.experimental.pallas import tpu_sc as plsc

_L = 16


def _xlane_sum(v):
    lanes = lax.iota(jnp.int32, _L)
    for sh in (8, 4, 2, 1):
        idx = (lanes + sh) & (_L - 1)
        v = v + v.at[idx].get(mode="promise_in_bounds")
    return v


def _rsqrt(x):
    i = lax.bitcast_convert_type(x, jnp.int32)
    y = lax.bitcast_convert_type(jnp.int32(0x5F3759DF) - (i >> 1), jnp.float32)
    for _ in range(3):
        y = y * (1.5 - 0.5 * x * y * y)
    return y


def kernel(input_ids, token_table, pos_table, ln_gamma, ln_beta):
    B, S = input_ids.shape
    V, H = token_table.shape
    n_vec = H // _L

    info = plsc.get_sparse_core_info()
    NC, NS = info.num_cores, info.num_subcores
    NW = NC * NS
    P = S // NW
    C = 16
    n_chunks = P // C

    mesh = plsc.VectorSubcoreMesh(core_axis_name="c", subcore_axis_name="s")

    @functools.partial(
        pl.kernel,
        mesh=mesh,
        out_type=jax.ShapeDtypeStruct((B, S, H), jnp.float32),
        scratch_types=[
            pltpu.VMEM((B, P), jnp.int32),
            pltpu.VMEM((2, B, C, H), jnp.float32),
            pltpu.VMEM((2, C, H), jnp.float32),
            pltpu.VMEM((H,), jnp.float32),
            pltpu.VMEM((H,), jnp.float32),
            pltpu.SemaphoreType.DMA,
            pltpu.SemaphoreType.DMA,
            pltpu.SemaphoreType.DMA,
            pltpu.SemaphoreType.DMA,
        ],
    )
    def emb_ln(ids_hbm, tok_hbm, pos_hbm, gam_hbm, bet_hbm, out_hbm,
               idx_v, rows_v, pos_v, gam_v, bet_v,
               isem0, isem1, osem0, osem1):
        isem = (isem0, isem1)
        osem = (osem0, osem1)
        wid = lax.axis_index("s") * NC + lax.axis_index("c")
        p_base = wid * P

        def fire_in(ci, p):
            p0 = p_base + ci * C
            pltpu.async_copy(pos_hbm.at[pl.ds(p0, C), :], pos_v.at[p],
                             isem[p])
            for b in range(B):
                pltpu.async_copy(tok_hbm.at[idx_v.at[b, pl.ds(ci * C, C)]],
                                 rows_v.at[p, b], isem[p])

        def wait_in(p):
            pltpu.make_async_copy(pos_hbm.at[pl.ds(0, C), :], pos_v.at[p],
                                  isem[p]).wait()
            for b in range(B):
                pltpu.make_async_copy(tok_hbm.at[pl.ds(0, C), :],
                                      rows_v.at[p, b], isem[p]).wait()

        def fire_out(ci, p):
            p0 = p_base + ci * C
            for b in range(B):
                pltpu.async_copy(rows_v.at[p, b],
                                 out_hbm.at[b, pl.ds(p0, C), :], osem[p])

        def wait_out(p):
            for b in range(B):
                pltpu.make_async_copy(rows_v.at[p, b],
                                      out_hbm.at[b, pl.ds(0, C), :],
                                      osem[p]).wait()

        def compute_chunk(rows_b, pos_b):
            def tok_body(t, _):
                for b in range(B):
                    zero = jnp.zeros((_L,), jnp.float32)
                    s = zero
                    q = zero
                    xs = []
                    for i in range(n_vec):
                        off = i * _L
                        v = (rows_b[b, t, pl.ds(off, _L)]
                             + pos_b[t, pl.ds(off, _L)])
                        xs.append(v)
                        s = s + v
                        q = q + v * v
                    mean = _xlane_sum(s) * (1.0 / H)
                    var = _xlane_sum(q) * (1.0 / H) - mean * mean
                    r = _rsqrt(var + 1e-5)
                    for i in range(n_vec):
                        off = i * _L
                        rows_b[b, t, pl.ds(off, _L)] = (
                            (xs[i] - mean) * r * gam_v[pl.ds(off, _L)]
                            + bet_v[pl.ds(off, _L)])
                return 0

            lax.fori_loop(0, C, tok_body, 0)

        for b in range(B):
            pltpu.sync_copy(ids_hbm.at[b, pl.ds(p_base, P)], idx_v.at[b])
        pltpu.sync_copy(gam_hbm, gam_v)
        pltpu.sync_copy(bet_hbm, bet_v)
        fire_in(0, 0)

        def outer(i, _):
            for par in (0, 1):
                ci = 2 * i + par

                @pl.when(ci > 0)
                def _():
                    wait_out(1 - par)

                @pl.when(ci + 1 < n_chunks)
                def _():
                    fire_in(ci + 1, 1 - par)

                wait_in(par)
                compute_chunk(rows_v.at[par], pos_v.at[par])
                fire_out(ci, par)
            return 0

        lax.fori_loop(0, n_chunks // 2, outer, 0)
        wait_out((n_chunks - 1) % 2)

    return emb_ln(input_ids.astype(jnp.int32), token_table, pos_table,
                  ln_gamma, ln_beta)
